# Initial kernel scaffold; baseline (speedup 1.0000x reference)
import jax
import jax.numpy as jnp
from jax.experimental import pallas as pl


def kernel(nodes, W_emb, b_emb, W1, A1, bn1_scale, bn1_bias, W2, A2, bn2_scale, bn2_bias, W3, A3, W_dec, b_dec, senders, receivers):
    raise NotImplementedError("write your pallas kernel here")


# speedup vs baseline: 42.9664x; 42.9664x over previous
import functools

import jax
import jax.numpy as jnp
from jax import lax
from jax.experimental import pallas as pl
from jax.experimental.pallas import tpu as pltpu
from jax.experimental.pallas import tpu_sc as plsc

N = 10000
E = 320000
D = 128
NB = 512
NB2 = NB * NB
NPAD = 10240
NC = 2
NS = 16
NW = NC * NS
EPW = E // NW
CH = 400
NCHUNK = EPW // CH
NGRP = CH // 16
CHB = 80
NCHUNKB = EPW // CHB
NGRPB = CHB // 16
ZSL = NPAD // NS
PSL = NB2 // NS

_f32 = jnp.float32
_i32 = jnp.int32


def _leaky(x):
    return jnp.where(x >= 0, x, 0.01 * x)


def _tc1_body(nodes_ref, wemb_ref, bemb_ref, w1_ref, a1_ref,
              h0_ref, hw1_ref, as_ref, ar_ref, bid_ref):
    nodes = nodes_ref[...]
    h0 = jnp.dot(nodes, wemb_ref[...], preferred_element_type=_f32) + bemb_ref[...]
    h0_ref[...] = h0
    hw1 = jnp.dot(h0, w1_ref[...], preferred_element_type=_f32)
    hw1_ref[...] = hw1
    a1 = a1_ref[...]
    as_ref[...] = jnp.dot(hw1, a1[:D, :], preferred_element_type=_f32)
    ar_ref[...] = jnp.dot(hw1, a1[D:, :], preferred_element_type=_f32)
    coords = nodes[:, 0:3]
    mn = jnp.min(coords, axis=0, keepdims=True)
    mx = jnp.max(coords, axis=0, keepdims=True)
    cell = (mx - mn) / 8.0
    gi = jnp.floor((coords - mn) / cell).astype(_i32)
    gi = jnp.clip(gi, 0, 7)
    bid = gi[:, 0:1] * 64 + gi[:, 1:2] * 8 + gi[:, 2:3]
    bid_ref[...] = bid


_tc1 = pl.pallas_call(
    _tc1_body,
    out_shape=[
        jax.ShapeDtypeStruct((N, D), _f32),
        jax.ShapeDtypeStruct((N, D), _f32),
        jax.ShapeDtypeStruct((N, 1), _f32),
        jax.ShapeDtypeStruct((N, 1), _f32),
        jax.ShapeDtypeStruct((N, 1), _i32),
    ],
)


_sc_mesh = plsc.VectorSubcoreMesh(core_axis_name="c", subcore_axis_name="s")
_sc_params = pltpu.CompilerParams(
    needs_layout_passes=False, use_tc_tiling_on_sc=False)


@functools.partial(
    pl.kernel,
    mesh=_sc_mesh,
    compiler_params=_sc_params,
    out_type=[
        jax.ShapeDtypeStruct((NC, NPAD), _f32),
        jax.ShapeDtypeStruct((NC, NPAD), _f32),
        jax.ShapeDtypeStruct((NC, NB2), _f32),
        jax.ShapeDtypeStruct((E,), _f32),
    ],
    scratch_types=[
        pltpu.VMEM((N,), _f32),
        pltpu.VMEM((N,), _f32),
        pltpu.VMEM((N,), _i32),
        pltpu.VMEM((CH,), _i32),
        pltpu.VMEM((CH,), _i32),
        pltpu.VMEM((CH,), _f32),
        pltpu.VMEM((CH,), _f32),
        pltpu.VMEM((CH,), _i32),
        pltpu.VMEM((1024,), _f32),
        pltpu.VMEM_SHARED((NPAD,), _f32),
        pltpu.VMEM_SHARED((NPAD,), _f32),
        pltpu.VMEM_SHARED((NB2,), _f32),
    ],
)
def _sca(as_hbm, ar_hbm, bid_hbm, snd_hbm, rcv_hbm,
         z_out, c_out, p_out, e_out,
         as_v, ar_v, bid_v, sidx_v, ridx_v, e_v, one_v, code_v, zero_v,
         z_sp, c_sp, p_sp):
    cc = lax.axis_index("c")
    ss = lax.axis_index("s")
    wid = ss * NC + cc

    zeros16 = jnp.zeros((16,), _f32)
    ones16 = jnp.ones((16,), _f32)

    def fill_zero(i, _):
        zero_v[pl.ds(i * 16, 16)] = zeros16
        return 0
    lax.fori_loop(0, 64, fill_zero, 0)

    def fill_one(i, _):
        one_v[pl.ds(i * 16, 16)] = ones16
        return 0
    lax.fori_loop(0, NGRP, fill_one, 0)

    pltpu.sync_copy(zero_v.at[pl.ds(0, ZSL)], z_sp.at[pl.ds(ss * ZSL, ZSL)])
    pltpu.sync_copy(zero_v.at[pl.ds(0, ZSL)], c_sp.at[pl.ds(ss * ZSL, ZSL)])

    def zero_p(i, _):
        pltpu.sync_copy(zero_v.at[pl.ds(0, 1024)],
                        p_sp.at[pl.ds(ss * PSL + i * 1024, 1024)])
        return 0
    lax.fori_loop(0, PSL // 1024, zero_p, 0)

    pltpu.sync_copy(as_hbm, as_v)
    pltpu.sync_copy(ar_hbm, ar_v)
    pltpu.sync_copy(bid_hbm, bid_v)

    plsc.subcore_barrier()

    def chunk_body(chi, _):
        base = wid * EPW + chi * CH
        pltpu.sync_copy(snd_hbm.at[pl.ds(base, CH)], sidx_v)
        pltpu.sync_copy(rcv_hbm.at[pl.ds(base, CH)], ridx_v)

        def grp(g, _):
            sl = pl.ds(g * 16, 16)
            s16 = sidx_v[sl]
            r16 = ridx_v[sl]
            av = plsc.load_gather(as_v, [s16])
            rv = plsc.load_gather(ar_v, [r16])
            ev = jnp.exp(_leaky(av + rv))
            e_v[sl] = ev
            bs = plsc.load_gather(bid_v, [s16])
            br = plsc.load_gather(bid_v, [r16])
            code = jnp.where(bs != br, bs * NB + br, NB2 - 1)
            code_v[sl] = code
            return 0
        lax.fori_loop(0, NGRP, grp, 0)

        pltpu.sync_copy(e_v, z_sp.at[ridx_v], add=True)
        pltpu.sync_copy(one_v, c_sp.at[ridx_v], add=True)
        pltpu.sync_copy(one_v, p_sp.at[code_v], add=True)
        pltpu.sync_copy(e_v, e_out.at[pl.ds(base, CH)])
        return 0
    lax.fori_loop(0, NCHUNK, chunk_body, 0)

    plsc.subcore_barrier()

    pltpu.sync_copy(z_sp.at[pl.ds(ss * ZSL, ZSL)],
                    z_out.at[cc, pl.ds(ss * ZSL, ZSL)])
    pltpu.sync_copy(c_sp.at[pl.ds(ss * ZSL, ZSL)],
                    c_out.at[cc, pl.ds(ss * ZSL, ZSL)])
    pltpu.sync_copy(p_sp.at[pl.ds(ss * PSL, PSL)],
                    p_out.at[cc, pl.ds(ss * PSL, PSL)])


@functools.partial(
    pl.kernel,
    mesh=_sc_mesh,
    compiler_params=_sc_params,
    out_type=[
        jax.ShapeDtypeStruct((NC, NPAD, D), _f32),
    ],
    scratch_types=[
        pltpu.VMEM((NPAD,), _f32),
        pltpu.VMEM((1024,), _f32),
        pltpu.VMEM((1024,), _f32),
        pltpu.VMEM((1024,), _f32),
        pltpu.VMEM((1024,), _f32),
        pltpu.VMEM((CHB,), _i32),
        pltpu.VMEM((CHB,), _i32),
        pltpu.VMEM((CHB,), _f32),
        pltpu.VMEM((CHB,), _f32),
        pltpu.VMEM((CHB, D), _f32),
        pltpu.VMEM_SHARED((NPAD, D), _f32),
        pltpu.SemaphoreType.DMA,
    ],
)
def _scb(z_hbm, c_hbm, e_hbm, snd_hbm, rcv_hbm, hw1_hbm,
         gout,
         f_v, zb0, zb1, cb0, cb1, sidx_v, ridx_v, e_v, w_v, rows_v,
         out_sp, sem):
    cc = lax.axis_index("c")
    ss = lax.axis_index("s")
    wid = ss * NC + cc

    def fchunk(i, _):
        off = i * 1024
        pltpu.sync_copy(z_hbm.at[0, pl.ds(off, 1024)], zb0)
        pltpu.sync_copy(z_hbm.at[1, pl.ds(off, 1024)], zb1)
        pltpu.sync_copy(c_hbm.at[0, pl.ds(off, 1024)], cb0)
        pltpu.sync_copy(c_hbm.at[1, pl.ds(off, 1024)], cb1)

        def fgrp(g, _):
            sl = pl.ds(g * 16, 16)
            zz = zb0[sl] + zb1[sl]
            ccnt = cb0[sl] + cb1[sl]
            f_v[pl.ds(off + g * 16, 16)] = 1.0 / (
                (zz + 1e-9) * jnp.maximum(ccnt, 1.0))
            return 0
        lax.fori_loop(0, 64, fgrp, 0)
        return 0
    lax.fori_loop(0, NPAD // 1024, fchunk, 0)

    zeros16 = jnp.zeros((16,), _f32)

    def zrow(r, _):
        for d in range(D // 16):
            rows_v[r, pl.ds(d * 16, 16)] = zeros16
        return 0
    lax.fori_loop(0, CHB, zrow, 0)

    def zout(i, _):
        pltpu.sync_copy(rows_v.at[pl.ds(0, CHB), :],
                        out_sp.at[pl.ds(ss * ZSL + i * CHB, CHB), :])
        return 0
    lax.fori_loop(0, ZSL // CHB, zout, 0)

    plsc.subcore_barrier()

    def chunk_body(chi, _):
        base = wid * EPW + chi * CHB
        pltpu.sync_copy(snd_hbm.at[pl.ds(base, CHB)], sidx_v)
        pltpu.sync_copy(rcv_hbm.at[pl.ds(base, CHB)], ridx_v)
        pltpu.sync_copy(e_hbm.at[pl.ds(base, CHB)], e_v)
        pltpu.async_copy(hw1_hbm.at[sidx_v], rows_v, sem).wait()

        def grp(g, _):
            sl = pl.ds(g * 16, 16)
            r16 = ridx_v[sl]
            w = e_v[sl] * plsc.load_gather(f_v, [r16])
            w_v[sl] = w
            for j in range(16):
                wj = plsc.load_gather(
                    w_v, [jnp.full((16,), g * 16 + j, _i32)])
                for d in range(D // 16):
                    dsl = pl.ds(d * 16, 16)
                    rows_v[g * 16 + j, dsl] = rows_v[g * 16 + j, dsl] * wj
            return 0
        lax.fori_loop(0, NGRPB, grp, 0)

        pltpu.sync_copy(rows_v, out_sp.at[ridx_v], add=True)
        return 0
    lax.fori_loop(0, NCHUNKB, chunk_body, 0)

    plsc.subcore_barrier()

    def wout(i, _):
        pltpu.sync_copy(out_sp.at[pl.ds(ss * ZSL + i * CHB, CHB), :],
                        gout.at[cc, pl.ds(ss * ZSL + i * CHB, CHB), :])
        return 0
    lax.fori_loop(0, ZSL // CHB, wout, 0)


def _tc2_body(h0_ref, gout_ref, bid_ref, p_ref,
              bn1s_ref, bn1b_ref, w2_ref, a2_ref, bn2s_ref, bn2b_ref,
              w3_ref, a3_ref, wdec_ref, bdec_ref, out_ref):
    g1 = gout_ref[0, :N, :] + gout_ref[1, :N, :]
    mu = jnp.mean(g1, axis=0, keepdims=True)
    var = jnp.mean(jnp.square(g1 - mu), axis=0, keepdims=True)
    b1 = (g1 - mu) / jnp.sqrt(var + 1e-5) * bn1s_ref[...] + bn1b_ref[...]
    h1 = jax.nn.silu(b1) + h0_ref[...]

    bid = bid_ref[...]

    hc_sum = jnp.zeros((NB, D), _f32)
    bc = jnp.zeros((1, NB), _f32)
    for i in range(N // 1000):
        rows = h1[i * 1000:(i + 1) * 1000, :]
        bch = bid[i * 1000:(i + 1) * 1000, :]
        onehot = (bch == lax.broadcasted_iota(_i32, (1000, NB), 1)).astype(_f32)
        hc_sum = hc_sum + jax.lax.dot_general(
            onehot, rows, (((0,), (0,)), ((), ())),
            preferred_element_type=_f32)
        bc = bc + jnp.sum(onehot, axis=0, keepdims=True)
    bcc = jnp.reshape(bc, (NB, 1))
    hc = hc_sum / jnp.maximum(bcc, 1.0)

    nb_val = jnp.max(bid) + 1
    vmask = (lax.broadcasted_iota(_i32, (NB, 1), 0) < nb_val).astype(_f32)
    nbf = nb_val.astype(_f32)

    pm = p_ref[0, :] + p_ref[1, :]
    pmat = jnp.reshape(pm, (NB, NB))
    notdiag = (lax.broadcasted_iota(_i32, (NB, NB), 0)
               != lax.broadcasted_iota(_i32, (NB, NB), 1))
    amask = jnp.logical_and(pmat > 0.0, notdiag)
    af = amask.astype(_f32)
    cnt = jnp.sum(af, axis=0, keepdims=True)
    inv_cnt = 1.0 / jnp.maximum(cnt, 1.0)

    def coarse_gat(hin, w_ref, a_ref):
        hw = jnp.dot(hin, w_ref[...], preferred_element_type=_f32)
        a = a_ref[...]
        a_s = jnp.dot(hw, a[:D, :], preferred_element_type=_f32)
        a_r = jnp.dot(hw, a[D:, :], preferred_element_type=_f32)
        s_mat = _leaky(a_s + jnp.reshape(a_r, (1, NB)))
        s_m = jnp.where(amask, s_mat, -1e30)
        m = jnp.max(s_m, axis=0, keepdims=True)
        e = jnp.where(amask, jnp.exp(s_mat - m), 0.0)
        zc = jnp.sum(e, axis=0, keepdims=True)
        coeff = e / (zc + 1e-9) * inv_cnt
        return jax.lax.dot_general(
            coeff, hw, (((0,), (0,)), ((), ())),
            preferred_element_type=_f32)

    g2 = coarse_gat(hc, w2_ref, a2_ref)
    mu2 = jnp.sum(g2 * vmask, axis=0, keepdims=True) / nbf
    var2 = jnp.sum(jnp.square(g2 - mu2) * vmask, axis=0, keepdims=True) / nbf
    b2 = (g2 - mu2) / jnp.sqrt(var2 + 1e-5) * bn2s_ref[...] + bn2b_ref[...]
    h2 = jax.nn.silu(b2) + hc

    g3 = coarse_gat(h2, w3_ref, a3_ref)
    h3 = g3 + h2

    agg = jnp.sum(h3 * vmask, axis=0, keepdims=True)
    out_ref[...] = (jnp.dot(agg, wdec_ref[...], preferred_element_type=_f32)
                    + bdec_ref[...])


_tc2 = pl.pallas_call(
    _tc2_body,
    out_shape=jax.ShapeDtypeStruct((1, 1), _f32),
)


def kernel(nodes, W_emb, b_emb, W1, A1, bn1_scale, bn1_bias,
           W2, A2, bn2_scale, bn2_bias, W3, A3, W_dec, b_dec,
           senders, receivers):
    h0, hw1, as2d, ar2d, bid2d = _tc1(
        nodes, W_emb, b_emb.reshape(1, D), W1, A1)
    as1 = as2d.reshape(N)
    ar1 = ar2d.reshape(N)
    bid1 = bid2d.reshape(N)
    z, c, p, e_all = _sca(as1, ar1, bid1, senders, receivers)
    (gout,) = _scb(z, c, e_all, senders, receivers, hw1)
    out = _tc2(h0, gout, bid2d, p,
               bn1_scale.reshape(1, D), bn1_bias.reshape(1, D),
               W2, A2, bn2_scale.reshape(1, D), bn2_bias.reshape(1, D),
               W3, A3, W_dec, b_dec.reshape(1, 1))
    return out


# speedup vs baseline: 62.9347x; 1.4647x over previous
import functools

import jax
import jax.numpy as jnp
from jax import lax
from jax.experimental import pallas as pl
from jax.experimental.pallas import tpu as pltpu
from jax.experimental.pallas import tpu_sc as plsc

N = 10000
E = 320000
D = 128
NB = 512
NB2 = NB * NB
NPAD = 10240
NC = 2
NS = 16
NW = NC * NS
EPW = E // NW
CH = 400
NCHUNK = EPW // CH
NGRP = CH // 16
CHB = 256
NFULLB = EPW // CHB
TAILB = EPW - NFULLB * CHB
NGRPB = CHB // 16
ZSL = NPAD // NS
PSL = NB2 // NS

_f32 = jnp.float32
_i32 = jnp.int32


def _leaky(x):
    return jnp.where(x >= 0, x, 0.01 * x)


def _tc1_body(nodes_ref, wemb_ref, bemb_ref, w1_ref, a1_ref,
              h0_ref, hw1_ref, as_ref, ar_ref, bid_ref):
    nodes = nodes_ref[...]
    h0 = jnp.dot(nodes, wemb_ref[...], preferred_element_type=_f32) + bemb_ref[...]
    h0_ref[...] = h0
    hw1 = jnp.dot(h0, w1_ref[...], preferred_element_type=_f32)
    hw1_ref[...] = hw1
    a1 = a1_ref[...]
    as_ref[...] = jnp.dot(hw1, a1[:D, :], preferred_element_type=_f32)
    ar_ref[...] = jnp.dot(hw1, a1[D:, :], preferred_element_type=_f32)
    coords = nodes[:, 0:3]
    mn = jnp.min(coords, axis=0, keepdims=True)
    mx = jnp.max(coords, axis=0, keepdims=True)
    cell = (mx - mn) / 8.0
    gi = jnp.floor((coords - mn) / cell).astype(_i32)
    gi = jnp.clip(gi, 0, 7)
    bid = gi[:, 0:1] * 64 + gi[:, 1:2] * 8 + gi[:, 2:3]
    bid_ref[...] = bid


_tc1 = pl.pallas_call(
    _tc1_body,
    out_shape=[
        jax.ShapeDtypeStruct((N, D), _f32),
        jax.ShapeDtypeStruct((N, D), _f32),
        jax.ShapeDtypeStruct((N, 1), _f32),
        jax.ShapeDtypeStruct((N, 1), _f32),
        jax.ShapeDtypeStruct((N, 1), _i32),
    ],
)


_sc_mesh = plsc.VectorSubcoreMesh(core_axis_name="c", subcore_axis_name="s")
_sc_params = pltpu.CompilerParams(
    needs_layout_passes=False, use_tc_tiling_on_sc=False)


@functools.partial(
    pl.kernel,
    mesh=_sc_mesh,
    compiler_params=_sc_params,
    out_type=[
        jax.ShapeDtypeStruct((NC, NPAD), _f32),
        jax.ShapeDtypeStruct((NC, NPAD), _f32),
        jax.ShapeDtypeStruct((NC, NB2), _f32),
        jax.ShapeDtypeStruct((E,), _f32),
    ],
    scratch_types=[
        pltpu.VMEM((N,), _f32),
        pltpu.VMEM((N,), _f32),
        pltpu.VMEM((N,), _i32),
        pltpu.VMEM((CH,), _i32),
        pltpu.VMEM((CH,), _i32),
        pltpu.VMEM((CH,), _f32),
        pltpu.VMEM((CH,), _f32),
        pltpu.VMEM((CH,), _i32),
        pltpu.VMEM((1024,), _f32),
        pltpu.VMEM_SHARED((NPAD,), _f32),
        pltpu.VMEM_SHARED((NPAD,), _f32),
        pltpu.VMEM_SHARED((NB2,), _f32),
    ],
)
def _sca(as_hbm, ar_hbm, bid_hbm, snd_hbm, rcv_hbm,
         z_out, c_out, p_out, e_out,
         as_v, ar_v, bid_v, sidx_v, ridx_v, e_v, one_v, code_v, zero_v,
         z_sp, c_sp, p_sp):
    cc = lax.axis_index("c")
    ss = lax.axis_index("s")
    wid = ss * NC + cc

    zeros16 = jnp.zeros((16,), _f32)
    ones16 = jnp.ones((16,), _f32)

    def fill_zero(i, _):
        zero_v[pl.ds(i * 16, 16)] = zeros16
        return 0
    lax.fori_loop(0, 64, fill_zero, 0)

    def fill_one(i, _):
        one_v[pl.ds(i * 16, 16)] = ones16
        return 0
    lax.fori_loop(0, NGRP, fill_one, 0)

    pltpu.sync_copy(zero_v.at[pl.ds(0, ZSL)], z_sp.at[pl.ds(ss * ZSL, ZSL)])
    pltpu.sync_copy(zero_v.at[pl.ds(0, ZSL)], c_sp.at[pl.ds(ss * ZSL, ZSL)])

    def zero_p(i, _):
        pltpu.sync_copy(zero_v.at[pl.ds(0, 1024)],
                        p_sp.at[pl.ds(ss * PSL + i * 1024, 1024)])
        return 0
    lax.fori_loop(0, PSL // 1024, zero_p, 0)

    pltpu.sync_copy(as_hbm, as_v)
    pltpu.sync_copy(ar_hbm, ar_v)
    pltpu.sync_copy(bid_hbm, bid_v)

    plsc.subcore_barrier()

    def chunk_body(chi, _):
        base = wid * EPW + chi * CH
        pltpu.sync_copy(snd_hbm.at[pl.ds(base, CH)], sidx_v)
        pltpu.sync_copy(rcv_hbm.at[pl.ds(base, CH)], ridx_v)

        def grp(g, _):
            sl = pl.ds(g * 16, 16)
            s16 = sidx_v[sl]
            r16 = ridx_v[sl]
            av = plsc.load_gather(as_v, [s16])
            rv = plsc.load_gather(ar_v, [r16])
            ev = jnp.exp(_leaky(av + rv))
            e_v[sl] = ev
            bs = plsc.load_gather(bid_v, [s16])
            br = plsc.load_gather(bid_v, [r16])
            code = jnp.where(bs != br, bs * NB + br, NB2 - 1)
            code_v[sl] = code
            return 0
        lax.fori_loop(0, NGRP, grp, 0)

        pltpu.sync_copy(e_v, z_sp.at[ridx_v], add=True)
        pltpu.sync_copy(one_v, c_sp.at[ridx_v], add=True)
        pltpu.sync_copy(one_v, p_sp.at[code_v], add=True)
        pltpu.sync_copy(e_v, e_out.at[pl.ds(base, CH)])
        return 0
    lax.fori_loop(0, NCHUNK, chunk_body, 0)

    plsc.subcore_barrier()

    pltpu.sync_copy(z_sp.at[pl.ds(ss * ZSL, ZSL)],
                    z_out.at[cc, pl.ds(ss * ZSL, ZSL)])
    pltpu.sync_copy(c_sp.at[pl.ds(ss * ZSL, ZSL)],
                    c_out.at[cc, pl.ds(ss * ZSL, ZSL)])
    pltpu.sync_copy(p_sp.at[pl.ds(ss * PSL, PSL)],
                    p_out.at[cc, pl.ds(ss * PSL, PSL)])


@functools.partial(
    pl.kernel,
    mesh=_sc_mesh,
    compiler_params=_sc_params,
    out_type=[
        jax.ShapeDtypeStruct((NC, NPAD, D), _f32),
    ],
    scratch_types=[
        pltpu.VMEM((NPAD,), _f32),
        pltpu.VMEM((1024,), _f32),
        pltpu.VMEM((1024,), _f32),
        pltpu.VMEM((CHB,), _i32),
        pltpu.VMEM((CHB,), _i32),
        pltpu.VMEM((CHB,), _f32),
        pltpu.VMEM((CHB,), _f32),
        pltpu.VMEM((CHB, D), _f32),
        pltpu.VMEM((TAILB,), _i32),
        pltpu.VMEM((TAILB,), _i32),
        pltpu.VMEM((TAILB,), _f32),
        pltpu.VMEM((TAILB, D), _f32),
        pltpu.VMEM_SHARED((NPAD, D), _f32),
        pltpu.SemaphoreType.DMA,
        pltpu.SemaphoreType.DMA,
    ],
)
def _scb(z_hbm, c_hbm, e_hbm, snd_hbm, rcv_hbm, hw1_hbm,
         gout,
         f_v, sb0, sb1, sidx_v, ridx_v, e_v, w_v, rows_v,
         sidx_t, ridx_t, e_t, rows_t,
         out_sp, gsem, ssem):
    cc = lax.axis_index("c")
    ss = lax.axis_index("s")
    wid = ss * NC + cc

    def fchunk_z(i, _):
        off = i * 1024
        pltpu.sync_copy(z_hbm.at[0, pl.ds(off, 1024)], sb0)
        pltpu.sync_copy(z_hbm.at[1, pl.ds(off, 1024)], sb1)

        def fgrp(g, _):
            sl16 = pl.ds(off + g * 16, 16)
            zz = sb0[pl.ds(g * 16, 16)] + sb1[pl.ds(g * 16, 16)]
            f_v[sl16] = 1.0 / (zz + 1e-9)
            return 0
        lax.fori_loop(0, 64, fgrp, 0)
        return 0
    lax.fori_loop(0, NPAD // 1024, fchunk_z, 0)

    def fchunk_c(i, _):
        off = i * 1024
        pltpu.sync_copy(c_hbm.at[0, pl.ds(off, 1024)], sb0)
        pltpu.sync_copy(c_hbm.at[1, pl.ds(off, 1024)], sb1)

        def fgrp(g, _):
            sl16 = pl.ds(off + g * 16, 16)
            ccnt = sb0[pl.ds(g * 16, 16)] + sb1[pl.ds(g * 16, 16)]
            f_v[sl16] = f_v[sl16] / jnp.maximum(ccnt, 1.0)
            return 0
        lax.fori_loop(0, 64, fgrp, 0)
        return 0
    lax.fori_loop(0, NPAD // 1024, fchunk_c, 0)

    zeros16 = jnp.zeros((16,), _f32)

    def zrow(r, _):
        for d in range(D // 16):
            rows_v[r, pl.ds(d * 16, 16)] = zeros16
        return 0
    lax.fori_loop(0, CHB, zrow, 0)

    def zout(i, _):
        pltpu.sync_copy(rows_v.at[pl.ds(0, 160), :],
                        out_sp.at[pl.ds(ss * ZSL + i * 160, 160), :])
        return 0
    lax.fori_loop(0, ZSL // 160, zout, 0)

    plsc.subcore_barrier()

    def scale_rows(rows_ref, w_ref, nrows):
        def row_body(j, _):
            wj = plsc.load_gather(w_ref, [jnp.full((16,), j, _i32)])
            for d in range(D // 16):
                dsl = pl.ds(d * 16, 16)
                rows_ref[j, dsl] = rows_ref[j, dsl] * wj
            return 0
        lax.fori_loop(0, nrows, row_body, 0, unroll=2)

    def chunk_body(chi, _):
        base = wid * EPW + chi * CHB
        pltpu.sync_copy(snd_hbm.at[pl.ds(base, CHB)], sidx_v)
        pltpu.sync_copy(rcv_hbm.at[pl.ds(base, CHB)], ridx_v)
        pltpu.sync_copy(e_hbm.at[pl.ds(base, CHB)], e_v)

        @pl.when(chi > 0)
        def _():
            pltpu.make_async_copy(
                rows_v, out_sp.at[pl.ds(0, CHB), :], ssem).wait()

        pltpu.async_copy(hw1_hbm.at[sidx_v], rows_v, gsem).wait()

        def grp(g, _):
            sl = pl.ds(g * 16, 16)
            r16 = ridx_v[sl]
            w_v[sl] = e_v[sl] * plsc.load_gather(f_v, [r16])
            return 0
        lax.fori_loop(0, NGRPB, grp, 0)

        scale_rows(rows_v, w_v, CHB)

        pltpu.async_copy(rows_v, out_sp.at[ridx_v], ssem, add=True)
        return 0
    lax.fori_loop(0, NFULLB, chunk_body, 0)

    pltpu.make_async_copy(rows_v, out_sp.at[pl.ds(0, CHB), :], ssem).wait()

    tbase = wid * EPW + NFULLB * CHB
    pltpu.sync_copy(snd_hbm.at[pl.ds(tbase, TAILB)], sidx_t)
    pltpu.sync_copy(rcv_hbm.at[pl.ds(tbase, TAILB)], ridx_t)
    pltpu.sync_copy(e_hbm.at[pl.ds(tbase, TAILB)], e_t)
    pltpu.async_copy(hw1_hbm.at[sidx_t], rows_t, gsem).wait()
    r16 = ridx_t[pl.ds(0, 16)]
    w_v[pl.ds(0, 16)] = e_t[pl.ds(0, 16)] * plsc.load_gather(f_v, [r16])
    scale_rows(rows_t, w_v, TAILB)
    pltpu.sync_copy(rows_t, out_sp.at[ridx_t], add=True)

    plsc.subcore_barrier()

    def wout(i, _):
        pltpu.sync_copy(out_sp.at[pl.ds(ss * ZSL + i * 320, 320), :],
                        gout.at[cc, pl.ds(ss * ZSL + i * 320, 320), :])
        return 0
    lax.fori_loop(0, ZSL // 320, wout, 0)


def _tc2_body(h0_ref, gout_ref, bid_ref, p_ref,
              bn1s_ref, bn1b_ref, w2_ref, a2_ref, bn2s_ref, bn2b_ref,
              w3_ref, a3_ref, wdec_ref, bdec_ref, out_ref):
    g1 = gout_ref[0, :N, :] + gout_ref[1, :N, :]
    mu = jnp.mean(g1, axis=0, keepdims=True)
    var = jnp.mean(jnp.square(g1 - mu), axis=0, keepdims=True)
    b1 = (g1 - mu) / jnp.sqrt(var + 1e-5) * bn1s_ref[...] + bn1b_ref[...]
    h1 = jax.nn.silu(b1) + h0_ref[...]

    bid = bid_ref[...]

    hc_sum = jnp.zeros((NB, D), _f32)
    bc = jnp.zeros((1, NB), _f32)
    for i in range(N // 1000):
        rows = h1[i * 1000:(i + 1) * 1000, :]
        bch = bid[i * 1000:(i + 1) * 1000, :]
        onehot = (bch == lax.broadcasted_iota(_i32, (1000, NB), 1)).astype(_f32)
        hc_sum = hc_sum + jax.lax.dot_general(
            onehot, rows, (((0,), (0,)), ((), ())),
            preferred_element_type=_f32)
        bc = bc + jnp.sum(onehot, axis=0, keepdims=True)
    bcc = jnp.reshape(bc, (NB, 1))
    hc = hc_sum / jnp.maximum(bcc, 1.0)

    nb_val = jnp.max(bid) + 1
    vmask = (lax.broadcasted_iota(_i32, (NB, 1), 0) < nb_val).astype(_f32)
    nbf = nb_val.astype(_f32)

    pm = p_ref[0, :] + p_ref[1, :]
    pmat = jnp.reshape(pm, (NB, NB))
    notdiag = (lax.broadcasted_iota(_i32, (NB, NB), 0)
               != lax.broadcasted_iota(_i32, (NB, NB), 1))
    amask = jnp.logical_and(pmat > 0.0, notdiag)
    af = amask.astype(_f32)
    cnt = jnp.sum(af, axis=0, keepdims=True)
    inv_cnt = 1.0 / jnp.maximum(cnt, 1.0)

    def coarse_gat(hin, w_ref, a_ref):
        hw = jnp.dot(hin, w_ref[...], preferred_element_type=_f32)
        a = a_ref[...]
        a_s = jnp.dot(hw, a[:D, :], preferred_element_type=_f32)
        a_r = jnp.dot(hw, a[D:, :], preferred_element_type=_f32)
        s_mat = _leaky(a_s + jnp.reshape(a_r, (1, NB)))
        s_m = jnp.where(amask, s_mat, -1e30)
        m = jnp.max(s_m, axis=0, keepdims=True)
        e = jnp.where(amask, jnp.exp(s_mat - m), 0.0)
        zc = jnp.sum(e, axis=0, keepdims=True)
        coeff = e / (zc + 1e-9) * inv_cnt
        return jax.lax.dot_general(
            coeff, hw, (((0,), (0,)), ((), ())),
            preferred_element_type=_f32)

    g2 = coarse_gat(hc, w2_ref, a2_ref)
    mu2 = jnp.sum(g2 * vmask, axis=0, keepdims=True) / nbf
    var2 = jnp.sum(jnp.square(g2 - mu2) * vmask, axis=0, keepdims=True) / nbf
    b2 = (g2 - mu2) / jnp.sqrt(var2 + 1e-5) * bn2s_ref[...] + bn2b_ref[...]
    h2 = jax.nn.silu(b2) + hc

    g3 = coarse_gat(h2, w3_ref, a3_ref)
    h3 = g3 + h2

    agg = jnp.sum(h3 * vmask, axis=0, keepdims=True)
    out_ref[...] = (jnp.dot(agg, wdec_ref[...], preferred_element_type=_f32)
                    + bdec_ref[...])


_tc2 = pl.pallas_call(
    _tc2_body,
    out_shape=jax.ShapeDtypeStruct((1, 1), _f32),
)


def kernel(nodes, W_emb, b_emb, W1, A1, bn1_scale, bn1_bias,
           W2, A2, bn2_scale, bn2_bias, W3, A3, W_dec, b_dec,
           senders, receivers):
    h0, hw1, as2d, ar2d, bid2d = _tc1(
        nodes, W_emb, b_emb.reshape(1, D), W1, A1)
    as1 = as2d.reshape(N)
    ar1 = ar2d.reshape(N)
    bid1 = bid2d.reshape(N)
    z, c, p, e_all = _sca(as1, ar1, bid1, senders, receivers)
    (gout,) = _scb(z, c, e_all, senders, receivers, hw1)
    out = _tc2(h0, gout, bid2d, p,
               bn1_scale.reshape(1, D), bn1_bias.reshape(1, D),
               W2, A2, bn2_scale.reshape(1, D), bn2_bias.reshape(1, D),
               W3, A3, W_dec, b_dec.reshape(1, 1))
    return out


# speedup vs baseline: 76.8382x; 1.2209x over previous
import functools

import jax
import jax.numpy as jnp
from jax import lax
from jax.experimental import pallas as pl
from jax.experimental.pallas import tpu as pltpu
from jax.experimental.pallas import tpu_sc as plsc

N = 10000
E = 320000
D = 128
NB = 512
NB2 = NB * NB
NPAD = 10240
NC = 2
NS = 16
NW = NC * NS
EPW = E // NW
CH = 400
NCHUNK = EPW // CH
NGRP = CH // 16
CHB = 80
NCHB = EPW // CHB
SUP = 400
NGRPB = CHB // 16
ZSL = NPAD // NS
PSL = NB2 // NS

_f32 = jnp.float32
_i32 = jnp.int32


def _leaky(x):
    return jnp.where(x >= 0, x, 0.01 * x)


def _tc1_body(nodes_ref, wemb_ref, bemb_ref, w1_ref, a1_ref,
              h0_ref, hw1_ref, as_ref, ar_ref, bid_ref):
    nodes = nodes_ref[...]
    h0 = jnp.dot(nodes, wemb_ref[...], preferred_element_type=_f32) + bemb_ref[...]
    h0_ref[...] = h0
    hw1 = jnp.dot(h0, w1_ref[...], preferred_element_type=_f32)
    hw1_ref[...] = hw1
    a1 = a1_ref[...]
    as_ref[...] = jnp.dot(hw1, a1[:D, :], preferred_element_type=_f32)
    ar_ref[...] = jnp.dot(hw1, a1[D:, :], preferred_element_type=_f32)
    coords = nodes[:, 0:3]
    mn = jnp.min(coords, axis=0, keepdims=True)
    mx = jnp.max(coords, axis=0, keepdims=True)
    cell = (mx - mn) / 8.0
    gi = jnp.floor((coords - mn) / cell).astype(_i32)
    gi = jnp.clip(gi, 0, 7)
    bid = gi[:, 0:1] * 64 + gi[:, 1:2] * 8 + gi[:, 2:3]
    bid_ref[...] = bid


_tc1 = pl.pallas_call(
    _tc1_body,
    out_shape=[
        jax.ShapeDtypeStruct((N, D), _f32),
        jax.ShapeDtypeStruct((N, D), _f32),
        jax.ShapeDtypeStruct((N, 1), _f32),
        jax.ShapeDtypeStruct((N, 1), _f32),
        jax.ShapeDtypeStruct((N, 1), _i32),
    ],
)


_sc_mesh = plsc.VectorSubcoreMesh(core_axis_name="c", subcore_axis_name="s")
_sc_params = pltpu.CompilerParams(
    needs_layout_passes=False, use_tc_tiling_on_sc=False)


@functools.partial(
    pl.kernel,
    mesh=_sc_mesh,
    compiler_params=_sc_params,
    out_type=[
        jax.ShapeDtypeStruct((NC, NPAD), _f32),
        jax.ShapeDtypeStruct((NC, NPAD), _f32),
        jax.ShapeDtypeStruct((NC, NB2), _f32),
        jax.ShapeDtypeStruct((E,), _f32),
    ],
    scratch_types=[
        pltpu.VMEM((N,), _f32),
        pltpu.VMEM((N,), _f32),
        pltpu.VMEM((N,), _i32),
        pltpu.VMEM((CH,), _i32),
        pltpu.VMEM((CH,), _i32),
        pltpu.VMEM((CH,), _f32),
        pltpu.VMEM((CH,), _f32),
        pltpu.VMEM((CH,), _i32),
        pltpu.VMEM((1024,), _f32),
        pltpu.VMEM_SHARED((NPAD,), _f32),
        pltpu.VMEM_SHARED((NPAD,), _f32),
        pltpu.VMEM_SHARED((NB2,), _f32),
    ],
)
def _sca(as_hbm, ar_hbm, bid_hbm, snd_hbm, rcv_hbm,
         z_out, c_out, p_out, e_out,
         as_v, ar_v, bid_v, sidx_v, ridx_v, e_v, one_v, code_v, zero_v,
         z_sp, c_sp, p_sp):
    cc = lax.axis_index("c")
    ss = lax.axis_index("s")
    wid = ss * NC + cc

    zeros16 = jnp.zeros((16,), _f32)
    ones16 = jnp.ones((16,), _f32)

    def fill_zero(i, _):
        zero_v[pl.ds(i * 16, 16)] = zeros16
        return 0
    lax.fori_loop(0, 64, fill_zero, 0)

    def fill_one(i, _):
        one_v[pl.ds(i * 16, 16)] = ones16
        return 0
    lax.fori_loop(0, NGRP, fill_one, 0)

    pltpu.sync_copy(zero_v.at[pl.ds(0, ZSL)], z_sp.at[pl.ds(ss * ZSL, ZSL)])
    pltpu.sync_copy(zero_v.at[pl.ds(0, ZSL)], c_sp.at[pl.ds(ss * ZSL, ZSL)])

    def zero_p(i, _):
        pltpu.sync_copy(zero_v.at[pl.ds(0, 1024)],
                        p_sp.at[pl.ds(ss * PSL + i * 1024, 1024)])
        return 0
    lax.fori_loop(0, PSL // 1024, zero_p, 0)

    pltpu.sync_copy(as_hbm, as_v)
    pltpu.sync_copy(ar_hbm, ar_v)
    pltpu.sync_copy(bid_hbm, bid_v)

    plsc.subcore_barrier()

    def chunk_body(chi, _):
        base = wid * EPW + chi * CH
        pltpu.sync_copy(snd_hbm.at[pl.ds(base, CH)], sidx_v)
        pltpu.sync_copy(rcv_hbm.at[pl.ds(base, CH)], ridx_v)

        def grp(g, _):
            sl = pl.ds(g * 16, 16)
            s16 = sidx_v[sl]
            r16 = ridx_v[sl]
            av = plsc.load_gather(as_v, [s16])
            rv = plsc.load_gather(ar_v, [r16])
            ev = jnp.exp(_leaky(av + rv))
            e_v[sl] = ev
            bs = plsc.load_gather(bid_v, [s16])
            br = plsc.load_gather(bid_v, [r16])
            code = jnp.where(bs != br, bs * NB + br, NB2 - 1)
            code_v[sl] = code
            return 0
        lax.fori_loop(0, NGRP, grp, 0)

        pltpu.sync_copy(e_v, z_sp.at[ridx_v], add=True)
        pltpu.sync_copy(one_v, c_sp.at[ridx_v], add=True)
        pltpu.sync_copy(one_v, p_sp.at[code_v], add=True)
        pltpu.sync_copy(e_v, e_out.at[pl.ds(base, CH)])
        return 0
    lax.fori_loop(0, NCHUNK, chunk_body, 0)

    plsc.subcore_barrier()

    pltpu.sync_copy(z_sp.at[pl.ds(ss * ZSL, ZSL)],
                    z_out.at[cc, pl.ds(ss * ZSL, ZSL)])
    pltpu.sync_copy(c_sp.at[pl.ds(ss * ZSL, ZSL)],
                    c_out.at[cc, pl.ds(ss * ZSL, ZSL)])
    pltpu.sync_copy(p_sp.at[pl.ds(ss * PSL, PSL)],
                    p_out.at[cc, pl.ds(ss * PSL, PSL)])


@functools.partial(
    pl.kernel,
    mesh=_sc_mesh,
    compiler_params=_sc_params,
    out_type=[
        jax.ShapeDtypeStruct((NC, NPAD, D), _f32),
    ],
    scratch_types=[
        pltpu.VMEM((NPAD,), _f32),
        pltpu.VMEM((1024,), _f32),
        pltpu.VMEM((1024,), _f32),
        pltpu.VMEM((SUP,), _i32),
        pltpu.VMEM((SUP,), _i32),
        pltpu.VMEM((SUP,), _f32),
        pltpu.VMEM((SUP,), _i32),
        pltpu.VMEM((SUP,), _i32),
        pltpu.VMEM((SUP,), _f32),
        pltpu.VMEM((SUP,), _f32),
        pltpu.VMEM((SUP,), _f32),
        pltpu.VMEM((CHB, D), _f32),
        pltpu.VMEM((CHB, D), _f32),
        pltpu.VMEM((CHB, D), _f32),
        pltpu.VMEM_SHARED((NPAD, D), _f32),
        pltpu.SemaphoreType.DMA,
        pltpu.SemaphoreType.DMA,
        pltpu.SemaphoreType.DMA,
        pltpu.SemaphoreType.DMA,
        pltpu.SemaphoreType.DMA,
        pltpu.SemaphoreType.DMA,
        pltpu.SemaphoreType.DMA,
        pltpu.SemaphoreType.DMA,
    ],
)
def _scb(z_hbm, c_hbm, e_hbm, snd_hbm, rcv_hbm, hw1_hbm,
         gout,
         f_v, sb0, sb1, s_sl0, r_sl0, e_sl0, s_sl1, r_sl1, e_sl1,
         w_sl0, w_sl1, rows0, rows1, rows2,
         out_sp, g0, g1, g2, s0, s1, s2, i0, i1):
    cc = lax.axis_index("c")
    ss = lax.axis_index("s")
    wid = ss * NC + cc

    def fchunk_z(i, _):
        off = i * 1024
        pltpu.sync_copy(z_hbm.at[0, pl.ds(off, 1024)], sb0)
        pltpu.sync_copy(z_hbm.at[1, pl.ds(off, 1024)], sb1)

        def fgrp(g, _):
            sl16 = pl.ds(off + g * 16, 16)
            zz = sb0[pl.ds(g * 16, 16)] + sb1[pl.ds(g * 16, 16)]
            f_v[sl16] = 1.0 / (zz + 1e-9)
            return 0
        lax.fori_loop(0, 64, fgrp, 0)
        return 0
    lax.fori_loop(0, NPAD // 1024, fchunk_z, 0)

    def fchunk_c(i, _):
        off = i * 1024
        pltpu.sync_copy(c_hbm.at[0, pl.ds(off, 1024)], sb0)
        pltpu.sync_copy(c_hbm.at[1, pl.ds(off, 1024)], sb1)

        def fgrp(g, _):
            sl16 = pl.ds(off + g * 16, 16)
            ccnt = sb0[pl.ds(g * 16, 16)] + sb1[pl.ds(g * 16, 16)]
            f_v[sl16] = f_v[sl16] / jnp.maximum(ccnt, 1.0)
            return 0
        lax.fori_loop(0, 64, fgrp, 0)
        return 0
    lax.fori_loop(0, NPAD // 1024, fchunk_c, 0)

    zeros16 = jnp.zeros((16,), _f32)

    def zrow(r, _):
        for d in range(D // 16):
            rows0[r, pl.ds(d * 16, 16)] = zeros16
        return 0
    lax.fori_loop(0, CHB, zrow, 0)

    def zout(i, _):
        pltpu.sync_copy(rows0.at[pl.ds(0, CHB), :],
                        out_sp.at[pl.ds(ss * ZSL + i * CHB, CHB), :])
        return 0
    lax.fori_loop(0, ZSL // CHB, zout, 0)

    plsc.subcore_barrier()

    rows = (rows0, rows1, rows2)
    gsems = (g0, g1, g2)
    ssems = (s0, s1, s2)
    slots = ((s_sl0, r_sl0, e_sl0, i0), (s_sl1, r_sl1, e_sl1, i1))
    wsls = (w_sl0, w_sl1)
    tbase = wid * EPW

    def compute_w(si):
        _, rref, eref, _ = slots[si & 1]
        wref = wsls[si & 1]

        def wg(g, _):
            r16 = rref[pl.ds(g * 16, 16)]
            wref[pl.ds(g * 16, 16)] = (
                eref[pl.ds(g * 16, 16)] * plsc.load_gather(f_v, [r16]))
            return 0
        lax.fori_loop(0, SUP // 16, wg, 0)

    def issue_idx(si):
        sref, rref, eref, isem = slots[si & 1]
        off = tbase + si * SUP
        pltpu.async_copy(snd_hbm.at[pl.ds(off, SUP)], sref, isem)
        pltpu.async_copy(rcv_hbm.at[pl.ds(off, SUP)], rref, isem)
        pltpu.async_copy(e_hbm.at[pl.ds(off, SUP)], eref, isem)

    def wait_idx(si):
        sref, rref, eref, isem = slots[si & 1]
        pltpu.make_async_copy(snd_hbm.at[pl.ds(0, SUP)], sref, isem).wait()
        pltpu.make_async_copy(rcv_hbm.at[pl.ds(0, SUP)], rref, isem).wait()
        pltpu.make_async_copy(e_hbm.at[pl.ds(0, SUP)], eref, isem).wait()

    def issue_gather(c):
        sref = slots[(c // 5) & 1][0]
        rb = c % 3
        pltpu.async_copy(
            hw1_hbm.at[sref.at[pl.ds((c % 5) * CHB, CHB)]],
            rows[rb], gsems[rb])

    def wait_gather(c):
        rb = c % 3
        pltpu.make_async_copy(
            hw1_hbm.at[pl.ds(0, CHB), :], rows[rb], gsems[rb]).wait()

    def issue_scatter(c):
        rref = slots[(c // 5) & 1][1]
        rb = c % 3
        pltpu.async_copy(
            rows[rb], out_sp.at[rref.at[pl.ds((c % 5) * CHB, CHB)]],
            ssems[rb], add=True)

    def wait_scatter(c):
        rb = c % 3
        pltpu.make_async_copy(
            rows[rb], out_sp.at[pl.ds(0, CHB), :], ssems[rb]).wait()

    sref0, rref0, eref0, isem0 = slots[0]
    pltpu.sync_copy(snd_hbm.at[pl.ds(tbase, SUP)], sref0)
    pltpu.sync_copy(rcv_hbm.at[pl.ds(tbase, SUP)], rref0)
    pltpu.sync_copy(e_hbm.at[pl.ds(tbase, SUP)], eref0)
    issue_gather(0)
    compute_w(0)

    for c in range(NCHB):
        k = c % 5
        rb = c % 3
        wref = wsls[(c // 5) & 1]
        off = k * CHB

        wait_gather(c)

        if c + 1 < NCHB:
            if c >= 2:
                wait_scatter(c + 1)
            if (c + 1) % 5 == 0:
                wait_idx((c + 1) // 5)
                compute_w((c + 1) // 5)
            issue_gather(c + 1)

        def row_body(j, _):
            wj = plsc.load_gather(wref, [jnp.full((16,), off + j, _i32)])

            def dmul(dd, _):
                dsl = pl.ds(dd * 16, 16)
                rows[rb][j, dsl] = rows[rb][j, dsl] * wj
                return 0
            lax.fori_loop(0, D // 16, dmul, 0)
            return 0
        lax.fori_loop(0, CHB, row_body, 0)

        issue_scatter(c)

        if k == 1 and (c // 5) + 1 < NCHB // 5:
            issue_idx((c // 5) + 1)

    wait_scatter(122)
    wait_scatter(123)
    wait_scatter(124)

    plsc.subcore_barrier()

    def wout(i, _):
        pltpu.sync_copy(out_sp.at[pl.ds(ss * ZSL + i * CHB, CHB), :],
                        gout.at[cc, pl.ds(ss * ZSL + i * CHB, CHB), :])
        return 0
    lax.fori_loop(0, ZSL // CHB, wout, 0)


def _tc2_body(h0_ref, gout_ref, bid_ref, p_ref,
              bn1s_ref, bn1b_ref, w2_ref, a2_ref, bn2s_ref, bn2b_ref,
              w3_ref, a3_ref, wdec_ref, bdec_ref, out_ref):
    g1 = gout_ref[0, :N, :] + gout_ref[1, :N, :]
    mu = jnp.mean(g1, axis=0, keepdims=True)
    var = jnp.mean(jnp.square(g1 - mu), axis=0, keepdims=True)
    b1 = (g1 - mu) / jnp.sqrt(var + 1e-5) * bn1s_ref[...] + bn1b_ref[...]
    h1 = jax.nn.silu(b1) + h0_ref[...]

    bid = bid_ref[...]

    hc_sum = jnp.zeros((NB, D), _f32)
    bc = jnp.zeros((1, NB), _f32)
    for i in range(N // 1000):
        rows = h1[i * 1000:(i + 1) * 1000, :]
        bch = bid[i * 1000:(i + 1) * 1000, :]
        onehot = (bch == lax.broadcasted_iota(_i32, (1000, NB), 1)).astype(_f32)
        hc_sum = hc_sum + jax.lax.dot_general(
            onehot, rows, (((0,), (0,)), ((), ())),
            preferred_element_type=_f32)
        bc = bc + jnp.sum(onehot, axis=0, keepdims=True)
    bcc = jnp.reshape(bc, (NB, 1))
    hc = hc_sum / jnp.maximum(bcc, 1.0)

    nb_val = jnp.max(bid) + 1
    vmask = (lax.broadcasted_iota(_i32, (NB, 1), 0) < nb_val).astype(_f32)
    nbf = nb_val.astype(_f32)

    pm = p_ref[0, :] + p_ref[1, :]
    pmat = jnp.reshape(pm, (NB, NB))
    notdiag = (lax.broadcasted_iota(_i32, (NB, NB), 0)
               != lax.broadcasted_iota(_i32, (NB, NB), 1))
    amask = jnp.logical_and(pmat > 0.0, notdiag)
    af = amask.astype(_f32)
    cnt = jnp.sum(af, axis=0, keepdims=True)
    inv_cnt = 1.0 / jnp.maximum(cnt, 1.0)

    def coarse_gat(hin, w_ref, a_ref):
        hw = jnp.dot(hin, w_ref[...], preferred_element_type=_f32)
        a = a_ref[...]
        a_s = jnp.dot(hw, a[:D, :], preferred_element_type=_f32)
        a_r = jnp.dot(hw, a[D:, :], preferred_element_type=_f32)
        s_mat = _leaky(a_s + jnp.reshape(a_r, (1, NB)))
        s_m = jnp.where(amask, s_mat, -1e30)
        m = jnp.max(s_m, axis=0, keepdims=True)
        e = jnp.where(amask, jnp.exp(s_mat - m), 0.0)
        zc = jnp.sum(e, axis=0, keepdims=True)
        coeff = e / (zc + 1e-9) * inv_cnt
        return jax.lax.dot_general(
            coeff, hw, (((0,), (0,)), ((), ())),
            preferred_element_type=_f32)

    g2 = coarse_gat(hc, w2_ref, a2_ref)
    mu2 = jnp.sum(g2 * vmask, axis=0, keepdims=True) / nbf
    var2 = jnp.sum(jnp.square(g2 - mu2) * vmask, axis=0, keepdims=True) / nbf
    b2 = (g2 - mu2) / jnp.sqrt(var2 + 1e-5) * bn2s_ref[...] + bn2b_ref[...]
    h2 = jax.nn.silu(b2) + hc

    g3 = coarse_gat(h2, w3_ref, a3_ref)
    h3 = g3 + h2

    agg = jnp.sum(h3 * vmask, axis=0, keepdims=True)
    out_ref[...] = (jnp.dot(agg, wdec_ref[...], preferred_element_type=_f32)
                    + bdec_ref[...])


_tc2 = pl.pallas_call(
    _tc2_body,
    out_shape=jax.ShapeDtypeStruct((1, 1), _f32),
)


def kernel(nodes, W_emb, b_emb, W1, A1, bn1_scale, bn1_bias,
           W2, A2, bn2_scale, bn2_bias, W3, A3, W_dec, b_dec,
           senders, receivers):
    h0, hw1, as2d, ar2d, bid2d = _tc1(
        nodes, W_emb, b_emb.reshape(1, D), W1, A1)
    as1 = as2d.reshape(N)
    ar1 = ar2d.reshape(N)
    bid1 = bid2d.reshape(N)
    z, c, p, e_all = _sca(as1, ar1, bid1, senders, receivers)
    (gout,) = _scb(z, c, e_all, senders, receivers, hw1)
    out = _tc2(h0, gout, bid2d, p,
               bn1_scale.reshape(1, D), bn1_bias.reshape(1, D),
               W2, A2, bn2_scale.reshape(1, D), bn2_bias.reshape(1, D),
               W3, A3, W_dec, b_dec.reshape(1, 1))
    return out
